# baseline (speedup 1.0000x reference)
import functools

import jax
import jax.numpy as jnp
from jax import lax
from jax.experimental import pallas as pl
from jax.experimental.pallas import tpu as pltpu
from jax.experimental.pallas import tpu_sc as plsc

B = 32
N = 16384
GS = 64
CHUNK_LO = (0, 22, 43)
CHUNK_HI = (22, 43, 64)
MAX_CHUNK_W = 22
L = 16
HALF = N // 2
UNROLL = 4
ZUNROLL = 16

_mesh = plsc.VectorSubcoreMesh(core_axis_name="c", subcore_axis_name="s")


@functools.partial(
    pl.kernel,
    mesh=_mesh,
    out_type=jax.ShapeDtypeStruct((B, GS * GS * GS), jnp.float32),
    scratch_types=[
        pltpu.VMEM((HALF,), jnp.float32),
        pltpu.VMEM((HALF,), jnp.float32),
        pltpu.VMEM((HALF,), jnp.float32),
        pltpu.VMEM((MAX_CHUNK_W * GS * GS,), jnp.float32),
        pltpu.SemaphoreType.DMA,
    ],
    compiler_params=pltpu.CompilerParams(
        needs_layout_passes=False,
        disable_bounds_checks=True,
    ),
)
def _splat(xs_hbm, ys_hbm, zs_hbm, out_hbm, xs_v, ys_v, zs_v, grid_v, sem):
    b = lax.axis_index("s") * 2 + lax.axis_index("c")

    zeros = jnp.zeros((L,), jnp.float32)

    def zero_body(i, carry):
        b0 = i * (L * ZUNROLL)
        for j in range(ZUNROLL):
            grid_v[pl.ds(b0 + j * L, L)] = zeros
        return carry

    def compute_scatters(qx, qy, qz, k):
        lo, hi = CHUNK_LO[k], CHUNK_HI[k]
        tix = qx.astype(jnp.int32)
        tiy = qy.astype(jnp.int32)
        tiz = qz.astype(jnp.int32)
        fx = qx - tix.astype(jnp.float32)
        fy = qy - tiy.astype(jnp.float32)
        fz = qz - tiz.astype(jnp.float32)
        wx0 = 1.0 - fx
        wy0 = 1.0 - fy
        wz0 = 1.0 - fz
        base = (tix << 12) + (tiy << 6) + tiz - (lo << 12)
        ixp = tix + 1
        if lo == 0:
            cx0 = tix < hi
            cx1 = ixp < hi
        elif hi == GS:
            cx0 = tix >= lo
            cx1 = jnp.logical_and(ixp >= lo, ixp <= GS - 1)
        else:
            cx0 = jnp.logical_and(tix >= lo, tix < hi)
            cx1 = jnp.logical_and(ixp >= lo, ixp < hi)
        my0 = tiy <= GS - 1
        mz0 = tiz <= GS - 1
        my1 = tiy <= GS - 2
        mz1 = tiz <= GS - 2
        m00 = jnp.logical_and(my0, mz0)
        m01 = jnp.logical_and(my0, mz1)
        m10 = jnp.logical_and(my1, mz0)
        m11 = jnp.logical_and(my1, mz1)
        wyz00 = wy0 * wz0
        wyz01 = wy0 * fz
        wyz10 = fy * wz0
        wyz11 = fy * fz
        return [
            (base, jnp.logical_and(cx0, m00), wx0 * wyz00),
            (base + 1, jnp.logical_and(cx0, m01), wx0 * wyz01),
            (base + GS, jnp.logical_and(cx0, m10), wx0 * wyz10),
            (base + GS + 1, jnp.logical_and(cx0, m11), wx0 * wyz11),
            (base + GS * GS, jnp.logical_and(cx1, m00), fx * wyz00),
            (base + GS * GS + 1, jnp.logical_and(cx1, m01), fx * wyz01),
            (base + GS * GS + GS, jnp.logical_and(cx1, m10), fx * wyz10),
            (base + GS * GS + GS + 1, jnp.logical_and(cx1, m11), fx * wyz11),
        ]

    out_off = 0
    for k in range(3):
        words = (CHUNK_HI[k] - CHUNK_LO[k]) * GS * GS
        lax.fori_loop(0, words // (L * ZUNROLL), zero_body, 0)
        for h in range(2):
            cx = pltpu.async_copy(xs_hbm.at[b, pl.ds(h * HALF, HALF)], xs_v, sem)
            cy = pltpu.async_copy(ys_hbm.at[b, pl.ds(h * HALF, HALF)], ys_v, sem)
            cz = pltpu.async_copy(zs_hbm.at[b, pl.ds(h * HALF, HALF)], zs_v, sem)
            cx.wait()
            cy.wait()
            cz.wait()

            def step(i, carry, k=k):
                coords = []
                for j in range(UNROLL):
                    s = pl.ds((i * UNROLL + j) * L, L)
                    coords.append(
                        (xs_v[s] * 32.0, ys_v[s] * 32.0, zs_v[s] * 32.0)
                    )
                scats = []
                for qx, qy, qz in coords:
                    scats += compute_scatters(qx, qy, qz, k)
                for idx, m, w in scats:
                    plsc.addupdate_scatter(grid_v, [idx], w, mask=m)
                return carry

            lax.fori_loop(0, HALF // (L * UNROLL), step, 0)
        plsc.subcore_barrier()
        pltpu.sync_copy(
            grid_v.at[pl.ds(0, words)], out_hbm.at[b, pl.ds(out_off, words)]
        )
        out_off += words


def kernel(ptcloud):
    shifted = ptcloud + 1.0
    xs = shifted[:, :, 0]
    ys = shifted[:, :, 1]
    zs = shifted[:, :, 2]
    return _splat(xs, ys, zs)


# speedup vs baseline: 1.2055x; 1.2055x over previous
import functools

import jax
import jax.numpy as jnp
from jax import lax
from jax.experimental import pallas as pl
from jax.experimental.pallas import tpu as pltpu
from jax.experimental.pallas import tpu_sc as plsc

B = 32
N = 16384
GS = 64
CHUNK_LO = (0, 22, 43)
CHUNK_HI = (22, 43, 64)
MAX_CHUNK_W = 22
GRID_WORDS = MAX_CHUNK_W * GS * GS + 512
L = 16
QUARTER = N // 4
NQ = 4
UNROLL = 4
ZUNROLL = 16

_mesh = plsc.VectorSubcoreMesh(core_axis_name="c", subcore_axis_name="s")


@functools.partial(
    pl.kernel,
    mesh=_mesh,
    out_type=jax.ShapeDtypeStruct((B, GS * GS * GS), jnp.float32),
    scratch_types=[
        pltpu.VMEM((QUARTER,), jnp.float32),
        pltpu.VMEM((QUARTER,), jnp.float32),
        pltpu.VMEM((QUARTER,), jnp.float32),
        pltpu.VMEM((QUARTER,), jnp.float32),
        pltpu.VMEM((QUARTER,), jnp.float32),
        pltpu.VMEM((QUARTER,), jnp.float32),
        pltpu.VMEM((GRID_WORDS,), jnp.float32),
        pltpu.SemaphoreType.DMA,
        pltpu.SemaphoreType.DMA,
    ],
    compiler_params=pltpu.CompilerParams(
        needs_layout_passes=False,
        disable_bounds_checks=True,
    ),
)
def _splat(
    xs_hbm, ys_hbm, zs_hbm, out_hbm,
    xs0, ys0, zs0, xs1, ys1, zs1, grid_v, sem0, sem1,
):
    b = lax.axis_index("s") * 2 + lax.axis_index("c")
    bufs = ((xs0, ys0, zs0, sem0), (xs1, ys1, zs1, sem1))

    def issue(q):
        xv, yv, zv, sem = bufs[q % 2]
        return (
            pltpu.async_copy(xs_hbm.at[b, pl.ds(q * QUARTER, QUARTER)], xv, sem),
            pltpu.async_copy(ys_hbm.at[b, pl.ds(q * QUARTER, QUARTER)], yv, sem),
            pltpu.async_copy(zs_hbm.at[b, pl.ds(q * QUARTER, QUARTER)], zv, sem),
        )

    zeros = jnp.zeros((L,), jnp.float32)

    def zero_body(i, carry):
        b0 = i * (L * ZUNROLL)
        for j in range(ZUNROLL):
            grid_v[pl.ds(b0 + j * L, L)] = zeros
        return carry

    def compute_scatters(qx, qy, qz, k):
        lo, hi = CHUNK_LO[k], CHUNK_HI[k]
        tix = qx.astype(jnp.int32)
        tiy = qy.astype(jnp.int32)
        tiz = qz.astype(jnp.int32)
        fx = qx - tix.astype(jnp.float32)
        fy = qy - tiy.astype(jnp.float32)
        fz = qz - tiz.astype(jnp.float32)
        wx0 = 1.0 - fx
        wy0 = jnp.where(tiy <= GS - 1, 1.0 - fy, 0.0)
        wz0 = jnp.where(tiz <= GS - 1, 1.0 - fz, 0.0)
        wy1 = jnp.where(tiy <= GS - 2, fy, 0.0)
        wz1 = jnp.where(tiz <= GS - 2, fz, 0.0)
        base = (tix << 12) + (tiy << 6) + tiz - (lo << 12)
        ixp = tix + 1
        if lo == 0:
            cx0 = tix < hi
            cx1 = ixp < hi
        elif hi == GS:
            cx0 = tix >= lo
            cx1 = jnp.logical_and(ixp >= lo, ixp <= GS - 1)
        else:
            cx0 = jnp.logical_and(tix >= lo, tix < hi)
            cx1 = jnp.logical_and(ixp >= lo, ixp < hi)
        wyz00 = wy0 * wz0
        wyz01 = wy0 * wz1
        wyz10 = wy1 * wz0
        wyz11 = wy1 * wz1
        return [
            (base, cx0, wx0 * wyz00),
            (base + 1, cx0, wx0 * wyz01),
            (base + GS, cx0, wx0 * wyz10),
            (base + GS + 1, cx0, wx0 * wyz11),
            (base + GS * GS, cx1, fx * wyz00),
            (base + GS * GS + 1, cx1, fx * wyz01),
            (base + GS * GS + GS, cx1, fx * wyz10),
            (base + GS * GS + GS + 1, cx1, fx * wyz11),
        ]

    out_off = 0
    for k in range(3):
        words = (CHUNK_HI[k] - CHUNK_LO[k]) * GS * GS
        pending = issue(0)
        lax.fori_loop(0, words // (L * ZUNROLL), zero_body, 0)
        for q in range(NQ):
            for c in pending:
                c.wait()
            if q < NQ - 1:
                pending = issue(q + 1)
            xv, yv, zv, _ = bufs[q % 2]

            def step(i, carry, k=k, xv=xv, yv=yv, zv=zv):
                coords = []
                for j in range(UNROLL):
                    s = pl.ds((i * UNROLL + j) * L, L)
                    coords.append((xv[s] * 32.0, yv[s] * 32.0, zv[s] * 32.0))
                scats = []
                for qx, qy, qz in coords:
                    scats += compute_scatters(qx, qy, qz, k)
                for idx, m, w in scats:
                    plsc.addupdate_scatter(grid_v, [idx], w, mask=m)
                return carry

            lax.fori_loop(0, QUARTER // (L * UNROLL), step, 0)
        plsc.subcore_barrier()
        pltpu.sync_copy(
            grid_v.at[pl.ds(0, words)], out_hbm.at[b, pl.ds(out_off, words)]
        )
        out_off += words


def kernel(ptcloud):
    shifted = ptcloud + 1.0
    xs = shifted[:, :, 0]
    ys = shifted[:, :, 1]
    zs = shifted[:, :, 2]
    return _splat(xs, ys, zs)


# speedup vs baseline: 1.2400x; 1.0287x over previous
import functools

import jax
import jax.numpy as jnp
from jax import lax
from jax.experimental import pallas as pl
from jax.experimental.pallas import tpu as pltpu
from jax.experimental.pallas import tpu_sc as plsc

B = 32
N = 16384
GS = 64
CHUNK_LO = (0, 22, 43)
CHUNK_HI = (22, 43, 64)
MAX_CHUNK_W = 22
GRID_WORDS = MAX_CHUNK_W * GS * GS + 512
L = 16
QUARTER = N // 4
NQ = 4
UNROLL = 8
ZUNROLL = 32

_mesh = plsc.VectorSubcoreMesh(core_axis_name="c", subcore_axis_name="s")


@functools.partial(
    pl.kernel,
    mesh=_mesh,
    out_type=jax.ShapeDtypeStruct((B, GS * GS * GS), jnp.float32),
    scratch_types=[
        pltpu.VMEM((QUARTER,), jnp.float32),
        pltpu.VMEM((QUARTER,), jnp.float32),
        pltpu.VMEM((QUARTER,), jnp.float32),
        pltpu.VMEM((QUARTER,), jnp.float32),
        pltpu.VMEM((QUARTER,), jnp.float32),
        pltpu.VMEM((QUARTER,), jnp.float32),
        pltpu.VMEM((GRID_WORDS,), jnp.float32),
        pltpu.SemaphoreType.DMA,
        pltpu.SemaphoreType.DMA,
    ],
    compiler_params=pltpu.CompilerParams(
        needs_layout_passes=False,
        disable_bounds_checks=True,
    ),
)
def _splat(
    xs_hbm, ys_hbm, zs_hbm, out_hbm,
    xs0, ys0, zs0, xs1, ys1, zs1, grid_v, sem0, sem1,
):
    b = lax.axis_index("s") * 2 + lax.axis_index("c")
    bufs = ((xs0, ys0, zs0, sem0), (xs1, ys1, zs1, sem1))

    def issue(q):
        xv, yv, zv, sem = bufs[q % 2]
        return (
            pltpu.async_copy(xs_hbm.at[b, pl.ds(q * QUARTER, QUARTER)], xv, sem),
            pltpu.async_copy(ys_hbm.at[b, pl.ds(q * QUARTER, QUARTER)], yv, sem),
            pltpu.async_copy(zs_hbm.at[b, pl.ds(q * QUARTER, QUARTER)], zv, sem),
        )

    zeros = jnp.zeros((L,), jnp.float32)

    def zero_body(i, carry):
        b0 = i * (L * ZUNROLL)
        for j in range(ZUNROLL):
            grid_v[pl.ds(b0 + j * L, L)] = zeros
        return carry

    def compute_scatters(qx, qy, qz, k):
        lo, hi = CHUNK_LO[k], CHUNK_HI[k]
        tix = qx.astype(jnp.int32)
        tiy = qy.astype(jnp.int32)
        tiz = qz.astype(jnp.int32)
        fx = qx - tix.astype(jnp.float32)
        fy = qy - tiy.astype(jnp.float32)
        fz = qz - tiz.astype(jnp.float32)
        wx0 = 1.0 - fx
        wy0 = jnp.where(tiy <= GS - 1, 1.0 - fy, 0.0)
        wz0 = jnp.where(tiz <= GS - 1, 1.0 - fz, 0.0)
        wy1 = jnp.where(tiy <= GS - 2, fy, 0.0)
        wz1 = jnp.where(tiz <= GS - 2, fz, 0.0)
        base = (tix << 12) + (tiy << 6) + tiz - (lo << 12)
        ixp = tix + 1
        if lo == 0:
            cx0 = tix < hi
            cx1 = ixp < hi
        elif hi == GS:
            cx0 = tix >= lo
            cx1 = jnp.logical_and(ixp >= lo, ixp <= GS - 1)
        else:
            cx0 = jnp.logical_and(tix >= lo, tix < hi)
            cx1 = jnp.logical_and(ixp >= lo, ixp < hi)
        wyz00 = wy0 * wz0
        wyz01 = wy0 * wz1
        wyz10 = wy1 * wz0
        wyz11 = wy1 * wz1
        return [
            (base, cx0, wx0 * wyz00),
            (base + 1, cx0, wx0 * wyz01),
            (base + GS, cx0, wx0 * wyz10),
            (base + GS + 1, cx0, wx0 * wyz11),
            (base + GS * GS, cx1, fx * wyz00),
            (base + GS * GS + 1, cx1, fx * wyz01),
            (base + GS * GS + GS, cx1, fx * wyz10),
            (base + GS * GS + GS + 1, cx1, fx * wyz11),
        ]

    out_off = 0
    for k in range(3):
        words = (CHUNK_HI[k] - CHUNK_LO[k]) * GS * GS
        pending = issue(0)
        lax.fori_loop(0, words // (L * ZUNROLL), zero_body, 0)
        for q in range(NQ):
            for c in pending:
                c.wait()
            if q < NQ - 1:
                pending = issue(q + 1)
            xv, yv, zv, _ = bufs[q % 2]

            def step(i, carry, k=k, xv=xv, yv=yv, zv=zv):
                coords = []
                for j in range(UNROLL):
                    s = pl.ds((i * UNROLL + j) * L, L)
                    coords.append((xv[s] * 32.0, yv[s] * 32.0, zv[s] * 32.0))
                scats = []
                for qx, qy, qz in coords:
                    scats += compute_scatters(qx, qy, qz, k)
                for idx, m, w in scats:
                    plsc.addupdate_scatter(grid_v, [idx], w, mask=m)
                return carry

            lax.fori_loop(0, QUARTER // (L * UNROLL), step, 0)
        plsc.subcore_barrier()
        pltpu.sync_copy(
            grid_v.at[pl.ds(0, words)], out_hbm.at[b, pl.ds(out_off, words)]
        )
        out_off += words


def kernel(ptcloud):
    shifted = ptcloud + 1.0
    xs = shifted[:, :, 0]
    ys = shifted[:, :, 1]
    zs = shifted[:, :, 2]
    return _splat(xs, ys, zs)
